# baseline (speedup 1.0000x reference)
import functools

import jax
import jax.numpy as jnp
import numpy as np
from jax import lax
from jax.experimental import pallas as pl
from jax.experimental.pallas import tpu as pltpu
from jax.experimental.pallas import tpu_sc as plsc

N = 100000
E = 1600000
NP = 100352
RPT = NP // 16
EP = 1605632
EROWS = EP // 128
KB = 28
B1 = EROWS // 32
B2 = EROWS // 16
BIAS = float(np.log(np.e - 1.0))
BN = 2048
GRID = NP // BN

_f32 = jnp.float32
_bf16 = jnp.bfloat16


def _zero_rows(zbuf, cols, dtype):
    def zrow(i, carry):
        zbuf[i, :] = jnp.zeros((cols,), dtype)
        return carry
    lax.fori_loop(0, 128, zrow, 0)


def _zero_acc_slice(acc, zbuf, base):
    def zcopy(k, carry):
        pltpu.sync_copy(zbuf, acc.at[pl.ds(base + k * 128, 128)])
        return carry
    lax.fori_loop(0, RPT // 128, zcopy, 0)


NGRP = 4
NG = KB // NGRP


def _edge_sweep(table, src_ids, dst_ids, acc, srcv, dstv, rows2, semg, sems,
                blk0, n_outer):
    def fire_gathers(bank, base):
        for k in range(NGRP):
            pltpu.async_copy(table.at[srcv.at[base + k]], rows2.at[bank, k],
                             semg)

    def drain_gathers(bank, base):
        for k in range(NGRP):
            pltpu.make_async_copy(table.at[srcv.at[base + k]],
                                  rows2.at[bank, k], semg).wait()

    def fire_scatters(bank, base):
        for k in range(NGRP):
            pltpu.async_copy(rows2.at[bank, k], acc.at[dstv.at[base + k]],
                             sems, add=True)

    def drain_scatters(bank, base):
        for k in range(NGRP):
            pltpu.make_async_copy(rows2.at[bank, k],
                                  acc.at[dstv.at[base + k]], sems).wait()

    def outer(ko, carry):
        r0 = blk0 + ko * KB
        pltpu.sync_copy(src_ids.at[pl.ds(r0, KB)], srcv)
        pltpu.sync_copy(dst_ids.at[pl.ds(r0, KB)], dstv)
        fire_gathers(0, 0)

        def grp(g, c2):
            bank = lax.rem(g, 2)
            base = g * NGRP
            drain_gathers(bank, base)
            fire_gathers(1 - bank, base + NGRP)
            fire_scatters(bank, base)
            drain_scatters(bank, base)
            return c2
        lax.fori_loop(0, NG - 1, grp, 0)
        last = (NG - 1) * NGRP
        lastbank = (NG - 1) % 2
        drain_gathers(lastbank, last)
        fire_scatters(lastbank, last)
        drain_scatters(lastbank, last)
        return carry
    lax.fori_loop(0, n_outer, outer, 0)


def _sc_layer1_body(xpad, src2d, dst2d, pout, acc, srcv, dstv, rows2, zbuf,
                    semg, sems):
    cid = lax.axis_index("c")
    sid = lax.axis_index("s")
    wid = sid * 2 + cid
    base = sid * RPT
    _zero_rows(zbuf, 16, _f32)
    _zero_acc_slice(acc, zbuf, base)
    plsc.subcore_barrier()
    _edge_sweep(xpad, src2d, dst2d, acc, srcv, dstv, rows2, semg, sems,
                wid * B1, B1 // KB)
    plsc.subcore_barrier()
    pltpu.sync_copy(acc.at[pl.ds(base, RPT)],
                    pout.at[pl.ds(cid * NP + base, RPT)])


def _sc_layer2_body(h2bf, src2, dst2d, aggout, acc, srcv, dstv, rows2, zbuf,
                    semg, sems):
    cid = lax.axis_index("c")
    sid = lax.axis_index("s")
    base = sid * RPT
    _zero_rows(zbuf, 32, _bf16)
    _zero_acc_slice(acc, zbuf, base)
    plsc.subcore_barrier()
    _edge_sweep(h2bf, src2.at[cid], dst2d, acc, srcv, dstv, rows2,
                semg, sems, sid * B2, B2 // KB)
    plsc.subcore_barrier()
    pltpu.sync_copy(acc.at[pl.ds(base, RPT)],
                    aggout.at[pl.ds(cid * NP + base, RPT)])


@functools.cache
def _get_sc_kernels():
    mesh = plsc.VectorSubcoreMesh(core_axis_name="c", subcore_axis_name="s",
                                  num_cores=2, num_subcores=16)
    scratch = [
        pltpu.VMEM_SHARED((NP, 16), _f32),
        pltpu.VMEM((KB, 128), jnp.int32),
        pltpu.VMEM((KB, 128), jnp.int32),
        pltpu.VMEM((2, NGRP, 128, 16), _f32),
        pltpu.VMEM((128, 16), _f32),
        pltpu.SemaphoreType.DMA,
        pltpu.SemaphoreType.DMA,
    ]
    params = pltpu.CompilerParams(use_tc_tiling_on_sc=False)
    sc1 = pl.kernel(
        _sc_layer1_body,
        out_type=jax.ShapeDtypeStruct((2 * NP, 16), _f32),
        mesh=mesh,
        scratch_types=scratch,
        compiler_params=params,
    )
    sc2 = pl.kernel(
        _sc_layer2_body,
        out_type=jax.ShapeDtypeStruct((2 * NP, 32), _bf16),
        mesh=mesh,
        scratch_types=[
            pltpu.VMEM_SHARED((NP, 32), _bf16),
            pltpu.VMEM((KB, 128), jnp.int32),
            pltpu.VMEM((KB, 128), jnp.int32),
            pltpu.VMEM((2, NGRP, 128, 32), _bf16),
            pltpu.VMEM((128, 32), _bf16),
            pltpu.SemaphoreType.DMA,
            pltpu.SemaphoreType.DMA,
        ],
        compiler_params=params,
    )
    return sc1, sc2


def _tc_l1_body(p_ref, x_ref, kwa_ref, kwb_ref, bias_ref, o_ref):
    p = p_ref[0] + p_ref[1]
    h2 = jnp.tanh(jnp.dot(p, kwa_ref[...], preferred_element_type=_f32)
                  + jnp.dot(x_ref[...], kwb_ref[...],
                            preferred_element_type=_f32)
                  + bias_ref[...])
    for p_ in range(2):
        words = []
        for k in range(8):
            lo = h2[:, 64 * k + 32 * p_:64 * k + 32 * p_ + 16]
            hi = h2[:, 64 * k + 32 * p_ + 16:64 * k + 32 * p_ + 32]
            lo16 = lax.bitcast_convert_type(
                lo.astype(_bf16), jnp.uint16).astype(jnp.uint32)
            hi16 = lax.bitcast_convert_type(
                hi.astype(_bf16), jnp.uint16).astype(jnp.uint32)
            words.append(lax.bitcast_convert_type(lo16 | (hi16 << 16),
                                                  jnp.int32))
        o_ref[p_] = jnp.concatenate(words, axis=1)


def _unpack_pair(w):
    u = lax.bitcast_convert_type(w, jnp.uint32)
    lo = lax.bitcast_convert_type(
        (u & jnp.uint32(0xFFFF)).astype(jnp.uint16), _bf16).astype(_f32)
    hi = lax.bitcast_convert_type(
        (u >> jnp.uint32(16)).astype(jnp.uint16), _bf16).astype(_f32)
    return lo, hi


def _tc_l2_body(a_ref, h_ref, kwr2_ref, kwo2_ref, kw2_ref, bias2_ref,
                b2t_ref, loc_ref, scale_ref, accs):
    i = pl.program_id(0)
    alo0, ahi0 = _unpack_pair(a_ref[0])
    alo1, ahi1 = _unpack_pair(a_ref[1])
    hlo0, hhi0 = _unpack_pair(h_ref[0])
    hlo1, hhi1 = _unpack_pair(h_ref[1])
    a_cat = jnp.concatenate([alo0, ahi0, alo1, ahi1], axis=1)
    h_cat = jnp.concatenate([hlo0, hhi0, hlo1, hhi1], axis=1)
    h3 = jnp.tanh(jnp.dot(a_cat, kwr2_ref[...], preferred_element_type=_f32)
                  + jnp.dot(h_cat, kwo2_ref[...],
                            preferred_element_type=_f32)
                  + bias2_ref[...])
    t = jnp.tanh(jnp.dot(h3, kw2_ref[...], preferred_element_type=_f32)
                 + b2t_ref[...])
    rows = (8 * lax.broadcasted_iota(jnp.int32, (BN // 8, 128), 0)
            + lax.div(lax.broadcasted_iota(jnp.int32, (BN // 8, 128), 1), 16)
            + i * BN)
    t = jnp.where(rows < N, t, 0.0)
    ps128 = jnp.sum(t, axis=0, keepdims=True)
    ps = sum(ps128[:, 16 * k:16 * (k + 1)] for k in range(8))

    @pl.when(i == 0)
    def _init():
        accs[...] = jnp.zeros_like(accs)

    accs[...] += ps

    @pl.when(i == GRID - 1)
    def _fini():
        pooled = accs[...] / _f32(N)
        loc_ref[...] = pooled[:, :8]
        sraw = pooled[:, 8:] + _f32(BIAS)
        sp = jnp.log1p(jnp.exp(sraw))
        scale_ref[...] = jnp.maximum(sp, _f32(1e-4))


_tc_l1 = pl.pallas_call(
    _tc_l1_body,
    grid=(GRID,),
    in_specs=[
        pl.BlockSpec((2, BN // 8, 128), lambda i: (0, i, 0)),
        pl.BlockSpec((BN // 8, 128), lambda i: (i, 0)),
        pl.BlockSpec((128, 512), lambda i: (0, 0)),
        pl.BlockSpec((128, 512), lambda i: (0, 0)),
        pl.BlockSpec((1, 512), lambda i: (0, 0)),
    ],
    out_specs=pl.BlockSpec((2, BN // 8, 128), lambda i: (0, i, 0)),
    out_shape=jax.ShapeDtypeStruct((2, NP // 8, 128), jnp.int32),
)

_tc_l2 = pl.pallas_call(
    _tc_l2_body,
    grid=(GRID,),
    in_specs=[
        pl.BlockSpec((2, BN // 8, 128), lambda i: (0, i, 0)),
        pl.BlockSpec((2, BN // 8, 128), lambda i: (0, i, 0)),
        pl.BlockSpec((512, 512), lambda i: (0, 0)),
        pl.BlockSpec((512, 512), lambda i: (0, 0)),
        pl.BlockSpec((512, 128), lambda i: (0, 0)),
        pl.BlockSpec((1, 512), lambda i: (0, 0)),
        pl.BlockSpec((1, 128), lambda i: (0, 0)),
    ],
    out_specs=[
        pl.BlockSpec((1, 8), lambda i: (0, 0)),
        pl.BlockSpec((1, 8), lambda i: (0, 0)),
    ],
    out_shape=[
        jax.ShapeDtypeStruct((1, 8), _f32),
        jax.ShapeDtypeStruct((1, 8), _f32),
    ],
    scratch_shapes=[pltpu.VMEM((1, 16), _f32)],
)


def kernel(x, W1, b1, Wr1, br1, Wo1, Wr2, br2, Wo2, W2, b2, edge_index):
    ei2 = edge_index.astype(jnp.int32).reshape(2, E // 128, 128)
    padrows = EROWS - E // 128
    src2d = jnp.pad(ei2[0], ((0, padrows), (0, 0)))
    dst2d = jnp.pad(ei2[1], ((0, padrows), (0, 0)), constant_values=N)
    src2 = src2d[None] + (jnp.arange(2, dtype=jnp.int32) * NP)[:, None, None]

    xp_in = jnp.pad(x, ((0, NP - N), (0, 0)))
    xpk = jnp.concatenate(
        [xp_in, jnp.ones((NP, 1), _f32), jnp.zeros((NP, 4), _f32)],
        axis=1).reshape(NP // 8, 128)
    xpad = xpk.reshape(NP, 16)

    eye8 = jnp.eye(8, dtype=_f32)
    w1pad = jnp.concatenate(
        [W1, b1[None, :], jnp.zeros((4, 64), _f32)], axis=0)
    wa = w1pad @ Wr1
    wb = w1pad @ Wo1
    kwa = jnp.einsum("kK,im->kiKm", eye8, wa).reshape(128, 512)
    kwb = jnp.einsum("kK,im->kiKm", eye8, wb).reshape(128, 512)
    bias1 = jnp.tile(br1[None, :], (1, 8))
    kwr2 = jnp.einsum("kK,cij->ckiKj", eye8,
                      Wr2.reshape(4, 16, 64)).reshape(512, 512)
    kwo2 = jnp.einsum("kK,cij->ckiKj", eye8,
                      Wo2.reshape(4, 16, 64)).reshape(512, 512)
    kw2 = jnp.einsum("kK,jm->kjKm", eye8, W2).reshape(512, 128)
    bias2 = jnp.tile(br2[None, :], (1, 8))
    b2t = jnp.tile(b2[None, :], (1, 8))

    _sc_layer1, _sc_layer2 = _get_sc_kernels()
    pflat = _sc_layer1(xpad, src2d, dst2d)
    p2 = pflat.reshape(2, NP // 8, 128)
    h2i = _tc_l1(p2, xpk, kwa, kwb, bias1)
    h2bf = lax.bitcast_convert_type(
        h2i.reshape(2 * NP, 16), _bf16).reshape(2 * NP, 32)
    aggbf = _sc_layer2(h2bf, src2, dst2d)
    aggi = lax.bitcast_convert_type(
        aggbf.reshape(2 * NP, 16, 2), jnp.int32).reshape(2, NP // 8, 128)
    loc2, scale2 = _tc_l2(aggi, h2i, kwr2, kwo2, kw2, bias2, b2t)
    return loc2.reshape(8), scale2.reshape(8)


# speedup vs baseline: 1.3548x; 1.3548x over previous
import functools

import jax
import jax.numpy as jnp
import numpy as np
from jax import lax
from jax.experimental import pallas as pl
from jax.experimental.pallas import tpu as pltpu
from jax.experimental.pallas import tpu_sc as plsc

N = 100000
E = 1600000
NP = 100352
RPT = NP // 16
EP = 1605632
EROWS = EP // 128
KB = 28
B1 = EROWS // 32
B2 = EROWS // 16
BIAS = float(np.log(np.e - 1.0))
BN = 2048
GRID = NP // BN

_f32 = jnp.float32
_bf16 = jnp.bfloat16


def _zero_rows(zbuf, cols, dtype):
    def zrow(i, carry):
        zbuf[i, :] = jnp.zeros((cols,), dtype)
        return carry
    lax.fori_loop(0, 128, zrow, 0)


def _zero_acc_slice(acc, zbuf, base):
    def zcopy(k, carry):
        pltpu.sync_copy(zbuf, acc.at[pl.ds(base + k * 128, 128)])
        return carry
    lax.fori_loop(0, RPT // 128, zcopy, 0)


NGRP = 4
NG = KB // NGRP


def _edge_sweep(table, src_ids, dst_ids, acc, srcv, dstv, rows2, semg, sems,
                blk0, n_outer, scatter_cast=None):
    def fire_gathers(bank, base):
        for k in range(NGRP):
            pltpu.async_copy(table.at[srcv.at[base + k]], rows2.at[bank, k],
                             semg)

    def drain_gathers(bank, base):
        for k in range(NGRP):
            pltpu.make_async_copy(table.at[srcv.at[base + k]],
                                  rows2.at[bank, k], semg).wait()

    def rowsrc(bank, k):
        r = rows2.at[bank, k]
        return r.bitcast(scatter_cast) if scatter_cast is not None else r

    def fire_scatters(bank, base):
        for k in range(NGRP):
            pltpu.async_copy(rowsrc(bank, k), acc.at[dstv.at[base + k]],
                             sems, add=True)

    def drain_scatters(bank, base):
        for k in range(NGRP):
            pltpu.make_async_copy(rowsrc(bank, k),
                                  acc.at[dstv.at[base + k]], sems).wait()

    def outer(ko, carry):
        r0 = blk0 + ko * KB
        pltpu.sync_copy(src_ids.at[pl.ds(r0, KB)], srcv)
        pltpu.sync_copy(dst_ids.at[pl.ds(r0, KB)], dstv)
        fire_gathers(0, 0)

        def grp(g, c2):
            bank = lax.rem(g, 2)
            base = g * NGRP
            drain_gathers(bank, base)
            fire_gathers(1 - bank, base + NGRP)
            fire_scatters(bank, base)
            drain_scatters(bank, base)
            return c2
        lax.fori_loop(0, NG - 1, grp, 0)
        last = (NG - 1) * NGRP
        lastbank = (NG - 1) % 2
        drain_gathers(lastbank, last)
        fire_scatters(lastbank, last)
        drain_scatters(lastbank, last)
        return carry
    lax.fori_loop(0, n_outer, outer, 0)


def _sc_layer1_body(xpad, src2d, dst2d, pout, acc, srcv, dstv, rows2, zbuf,
                    semg, sems):
    cid = lax.axis_index("c")
    sid = lax.axis_index("s")
    wid = sid * 2 + cid
    base = sid * RPT
    _zero_rows(zbuf, 16, _f32)
    _zero_acc_slice(acc, zbuf, base)
    plsc.subcore_barrier()
    _edge_sweep(xpad, src2d, dst2d, acc, srcv, dstv, rows2, semg, sems,
                wid * B1, B1 // KB)
    plsc.subcore_barrier()
    pltpu.sync_copy(acc.at[pl.ds(base, RPT)],
                    pout.at[pl.ds(cid * NP + base, RPT)])


def _sc_layer2_body(h2flat, src4, dst2d, aggout, acc, srcv, dstv, rows2, zbuf,
                    semg, sems):
    cid = lax.axis_index("c")
    sid = lax.axis_index("s")
    base = sid * RPT
    _zero_rows(zbuf, 16, _f32)

    def chunk_body(p, carry):
        c = cid * 2 + p
        _zero_acc_slice(acc, zbuf, base)
        plsc.subcore_barrier()
        _edge_sweep(h2flat, src4.at[c], dst2d, acc, srcv, dstv, rows2,
                    semg, sems, sid * B2, B2 // KB)
        plsc.subcore_barrier()
        pltpu.sync_copy(acc.at[pl.ds(base, RPT)],
                        aggout.at[pl.ds(c * NP + base, RPT)])
        plsc.subcore_barrier()
        return carry
    lax.fori_loop(0, 2, chunk_body, 0)


@functools.cache
def _get_sc_kernels():
    mesh = plsc.VectorSubcoreMesh(core_axis_name="c", subcore_axis_name="s",
                                  num_cores=2, num_subcores=16)
    scratch = [
        pltpu.VMEM_SHARED((NP, 16), _f32),
        pltpu.VMEM((KB, 128), jnp.int32),
        pltpu.VMEM((KB, 128), jnp.int32),
        pltpu.VMEM((2, NGRP, 128, 16), _f32),
        pltpu.VMEM((128, 16), _f32),
        pltpu.SemaphoreType.DMA,
        pltpu.SemaphoreType.DMA,
    ]
    params = pltpu.CompilerParams(use_tc_tiling_on_sc=False)
    sc1 = pl.kernel(
        _sc_layer1_body,
        out_type=jax.ShapeDtypeStruct((2 * NP, 16), _f32),
        mesh=mesh,
        scratch_types=scratch,
        compiler_params=params,
    )
    sc2 = pl.kernel(
        _sc_layer2_body,
        out_type=jax.ShapeDtypeStruct((4 * NP, 16), _f32),
        mesh=mesh,
        scratch_types=scratch,
        compiler_params=params,
    )
    return sc1, sc2


def _tc_l1_body(p_ref, x_ref, kwa_ref, kwb_ref, bias_ref, o_ref):
    p = p_ref[0] + p_ref[1]
    h2 = jnp.tanh(jnp.dot(p, kwa_ref[...], preferred_element_type=_f32)
                  + jnp.dot(x_ref[...], kwb_ref[...],
                            preferred_element_type=_f32)
                  + bias_ref[...])
    for c in range(4):
        o_ref[c] = jnp.concatenate(
            [h2[:, 64 * k + 16 * c:64 * k + 16 * (c + 1)] for k in range(8)],
            axis=1)


def _tc_l2_body(a_ref, h_ref, kwr2_ref, kwo2_ref, kw2_ref, bias2_ref,
                b2t_ref, loc_ref, scale_ref, accs):
    i = pl.program_id(0)
    a_cat = jnp.concatenate([a_ref[c] for c in range(4)], axis=1)
    h_cat = jnp.concatenate([h_ref[c] for c in range(4)], axis=1)
    h3 = jnp.tanh(jnp.dot(a_cat, kwr2_ref[...], preferred_element_type=_f32)
                  + jnp.dot(h_cat, kwo2_ref[...],
                            preferred_element_type=_f32)
                  + bias2_ref[...])
    t = jnp.tanh(jnp.dot(h3, kw2_ref[...], preferred_element_type=_f32)
                 + b2t_ref[...])
    rows = (8 * lax.broadcasted_iota(jnp.int32, (BN // 8, 128), 0)
            + lax.div(lax.broadcasted_iota(jnp.int32, (BN // 8, 128), 1), 16)
            + i * BN)
    t = jnp.where(rows < N, t, 0.0)
    ps128 = jnp.sum(t, axis=0, keepdims=True)
    ps = sum(ps128[:, 16 * k:16 * (k + 1)] for k in range(8))

    @pl.when(i == 0)
    def _init():
        accs[...] = jnp.zeros_like(accs)

    accs[...] += ps

    @pl.when(i == GRID - 1)
    def _fini():
        pooled = accs[...] / _f32(N)
        loc_ref[...] = pooled[:, :8]
        sraw = pooled[:, 8:] + _f32(BIAS)
        sp = jnp.log1p(jnp.exp(sraw))
        scale_ref[...] = jnp.maximum(sp, _f32(1e-4))


_tc_l1 = pl.pallas_call(
    _tc_l1_body,
    grid=(GRID,),
    in_specs=[
        pl.BlockSpec((2, BN // 8, 128), lambda i: (0, i, 0)),
        pl.BlockSpec((BN // 8, 128), lambda i: (i, 0)),
        pl.BlockSpec((128, 512), lambda i: (0, 0)),
        pl.BlockSpec((128, 512), lambda i: (0, 0)),
        pl.BlockSpec((1, 512), lambda i: (0, 0)),
    ],
    out_specs=pl.BlockSpec((4, BN // 8, 128), lambda i: (0, i, 0)),
    out_shape=jax.ShapeDtypeStruct((4, NP // 8, 128), _f32),
)

_tc_l2 = pl.pallas_call(
    _tc_l2_body,
    grid=(GRID,),
    in_specs=[
        pl.BlockSpec((4, BN // 8, 128), lambda i: (0, i, 0)),
        pl.BlockSpec((4, BN // 8, 128), lambda i: (0, i, 0)),
        pl.BlockSpec((512, 512), lambda i: (0, 0)),
        pl.BlockSpec((512, 512), lambda i: (0, 0)),
        pl.BlockSpec((512, 128), lambda i: (0, 0)),
        pl.BlockSpec((1, 512), lambda i: (0, 0)),
        pl.BlockSpec((1, 128), lambda i: (0, 0)),
    ],
    out_specs=[
        pl.BlockSpec((1, 8), lambda i: (0, 0)),
        pl.BlockSpec((1, 8), lambda i: (0, 0)),
    ],
    out_shape=[
        jax.ShapeDtypeStruct((1, 8), _f32),
        jax.ShapeDtypeStruct((1, 8), _f32),
    ],
    scratch_shapes=[pltpu.VMEM((1, 16), _f32)],
)


def kernel(x, W1, b1, Wr1, br1, Wo1, Wr2, br2, Wo2, W2, b2, edge_index):
    ei2 = edge_index.astype(jnp.int32).reshape(2, E // 128, 128)
    padrows = EROWS - E // 128
    src2d = jnp.pad(ei2[0], ((0, padrows), (0, 0)))
    dst2d = jnp.pad(ei2[1], ((0, padrows), (0, 0)), constant_values=N)
    src4 = src2d[None] + (jnp.arange(4, dtype=jnp.int32) * NP)[:, None, None]

    xp_in = jnp.pad(x, ((0, NP - N), (0, 0)))
    xpk = jnp.concatenate(
        [xp_in, jnp.ones((NP, 1), _f32), jnp.zeros((NP, 4), _f32)],
        axis=1).reshape(NP // 8, 128)
    xpad = xpk.reshape(NP, 16)

    eye8 = jnp.eye(8, dtype=_f32)
    w1pad = jnp.concatenate(
        [W1, b1[None, :], jnp.zeros((4, 64), _f32)], axis=0)
    wa = w1pad @ Wr1
    wb = w1pad @ Wo1
    kwa = jnp.einsum("kK,im->kiKm", eye8, wa).reshape(128, 512)
    kwb = jnp.einsum("kK,im->kiKm", eye8, wb).reshape(128, 512)
    bias1 = jnp.tile(br1[None, :], (1, 8))
    kwr2 = jnp.einsum("kK,cij->ckiKj", eye8,
                      Wr2.reshape(4, 16, 64)).reshape(512, 512)
    kwo2 = jnp.einsum("kK,cij->ckiKj", eye8,
                      Wo2.reshape(4, 16, 64)).reshape(512, 512)
    kw2 = jnp.einsum("kK,jm->kjKm", eye8, W2).reshape(512, 128)
    bias2 = jnp.tile(br2[None, :], (1, 8))
    b2t = jnp.tile(b2[None, :], (1, 8))

    _sc_layer1, _sc_layer2 = _get_sc_kernels()
    pflat = _sc_layer1(xpad, src2d, dst2d)
    p2 = pflat.reshape(2, NP // 8, 128)
    h2pk = _tc_l1(p2, xpk, kwa, kwb, bias1)
    aggflat = _sc_layer2(h2pk.reshape(4 * NP, 16), src4, dst2d)
    agg = aggflat.reshape(4, NP // 8, 128)
    loc2, scale2 = _tc_l2(agg, h2pk, kwr2, kwo2, kw2, bias2, b2t)
    return loc2.reshape(8), scale2.reshape(8)


# speedup vs baseline: 1.3965x; 1.0308x over previous
import functools

import jax
import jax.numpy as jnp
import numpy as np
from jax import lax
from jax.experimental import pallas as pl
from jax.experimental.pallas import tpu as pltpu
from jax.experimental.pallas import tpu_sc as plsc

N = 100000
E = 1600000
NP = 100352
RPT = NP // 16
EP = 1605632
EROWS = EP // 128
KB = 28
B1 = EROWS // 32
B2 = EROWS // 16
BIAS = float(np.log(np.e - 1.0))
BN = 2048
GRID = NP // BN

_f32 = jnp.float32
_bf16 = jnp.bfloat16


def _zero_rows(zbuf, cols, dtype):
    def zrow(i, carry):
        zbuf[i, :] = jnp.zeros((cols,), dtype)
        return carry
    lax.fori_loop(0, 128, zrow, 0)


def _zero_acc_slice(acc, zbuf, base):
    def zcopy(k, carry):
        pltpu.sync_copy(zbuf, acc.at[pl.ds(base + k * 128, 128)])
        return carry
    lax.fori_loop(0, RPT // 128, zcopy, 0)


NGRP = 4
NG = KB // NGRP


def _edge_sweep(table, src_ids, dst_ids, acc, srcv, dstv, rows2, semg, sems,
                blk0, n_outer, scatter_cast=None):
    def fire_gathers(bank, base):
        for k in range(NGRP):
            pltpu.async_copy(table.at[srcv.at[base + k]], rows2.at[bank, k],
                             semg)

    def drain_gathers(bank, base):
        for k in range(NGRP):
            pltpu.make_async_copy(table.at[srcv.at[base + k]],
                                  rows2.at[bank, k], semg).wait()

    def rowsrc(bank, k):
        r = rows2.at[bank, k]
        return r.bitcast(scatter_cast) if scatter_cast is not None else r

    def fire_scatters(bank, base):
        for k in range(NGRP):
            pltpu.async_copy(rowsrc(bank, k), acc.at[dstv.at[base + k]],
                             sems, add=True)

    def drain_scatters(bank, base):
        for k in range(NGRP):
            pltpu.make_async_copy(rowsrc(bank, k),
                                  acc.at[dstv.at[base + k]], sems).wait()

    def outer(ko, carry):
        r0 = blk0 + ko * KB
        pltpu.sync_copy(src_ids.at[pl.ds(r0, KB)], srcv)
        pltpu.sync_copy(dst_ids.at[pl.ds(r0, KB)], dstv)
        fire_gathers(0, 0)

        def grp(g, c2):
            bank = lax.rem(g, 2)
            base = g * NGRP
            drain_gathers(bank, base)
            fire_gathers(1 - bank, base + NGRP)
            fire_scatters(bank, base)
            drain_scatters(bank, base)
            return c2
        lax.fori_loop(0, NG - 1, grp, 0)
        last = (NG - 1) * NGRP
        lastbank = (NG - 1) % 2
        drain_gathers(lastbank, last)
        fire_scatters(lastbank, last)
        drain_scatters(lastbank, last)
        return carry
    lax.fori_loop(0, n_outer, outer, 0)


def _sc_layer1_body(xpad, src2d, dst2d, pout, acc, srcv, dstv, rows2, zbuf,
                    semg, sems):
    cid = lax.axis_index("c")
    sid = lax.axis_index("s")
    wid = sid * 2 + cid
    base = sid * RPT
    _zero_rows(zbuf, 16, _f32)
    _zero_acc_slice(acc, zbuf, base)
    plsc.subcore_barrier()
    _edge_sweep(xpad, src2d, dst2d, acc, srcv, dstv, rows2, semg, sems,
                wid * B1, B1 // KB)
    plsc.subcore_barrier()
    pltpu.sync_copy(acc.at[pl.ds(base, RPT)],
                    pout.at[pl.ds(cid * NP + base, RPT)])


def _sc_layer2_body(h2flat, src4, dst2d, aggout, acc, srcv, dstv, rows2, zbuf,
                    semg, sems):
    cid = lax.axis_index("c")
    sid = lax.axis_index("s")
    base = sid * RPT
    _zero_rows(zbuf, 16, _f32)

    def chunk_body(p, carry):
        c = cid * 2 + p
        _zero_acc_slice(acc, zbuf, base)
        plsc.subcore_barrier()
        _edge_sweep(h2flat, src4.at[c], dst2d, acc, srcv, dstv, rows2,
                    semg, sems, sid * B2, B2 // KB)
        plsc.subcore_barrier()
        pltpu.sync_copy(acc.at[pl.ds(base, RPT)],
                        aggout.at[pl.ds(c * NP + base, RPT)])
        plsc.subcore_barrier()
        return carry
    lax.fori_loop(0, 2, chunk_body, 0)


@functools.cache
def _get_sc_kernels():
    mesh = plsc.VectorSubcoreMesh(core_axis_name="c", subcore_axis_name="s",
                                  num_cores=2, num_subcores=16)
    scratch = [
        pltpu.VMEM_SHARED((NP, 16), _f32),
        pltpu.VMEM((KB, 128), jnp.int32),
        pltpu.VMEM((KB, 128), jnp.int32),
        pltpu.VMEM((2, NGRP, 128, 16), _f32),
        pltpu.VMEM((128, 16), _f32),
        pltpu.SemaphoreType.DMA,
        pltpu.SemaphoreType.DMA,
    ]
    params = pltpu.CompilerParams(use_tc_tiling_on_sc=False)
    sc1 = pl.kernel(
        _sc_layer1_body,
        out_type=jax.ShapeDtypeStruct((2 * NP, 16), _f32),
        mesh=mesh,
        scratch_types=scratch,
        compiler_params=params,
    )
    sc2 = pl.kernel(
        _sc_layer2_body,
        out_type=jax.ShapeDtypeStruct((4 * NP, 16), _f32),
        mesh=mesh,
        scratch_types=scratch,
        compiler_params=params,
    )
    return sc1, sc2


EB = 128


def _tc_prep_body(ei_ref, src_ref, dst_ref, src4_ref):
    i = pl.program_id(0)
    rows = lax.broadcasted_iota(jnp.int32, (EB, 128), 0) + i * EB
    valid = rows < (E // 128)
    s = jnp.where(valid, ei_ref[0], 0)
    dst_ref[...] = jnp.where(valid, ei_ref[1], N)
    src_ref[...] = s
    for c in range(4):
        src4_ref[c] = s + c * NP


def _tc_l1_body(p_ref, x_ref, kwa_ref, kwb_ref, bias_ref, o_ref):
    p = p_ref[0] + p_ref[1]
    h2 = jnp.tanh(jnp.dot(p, kwa_ref[...], preferred_element_type=_f32)
                  + jnp.dot(x_ref[...], kwb_ref[...],
                            preferred_element_type=_f32)
                  + bias_ref[...])
    for c in range(4):
        o_ref[c] = jnp.concatenate(
            [h2[:, 64 * k + 16 * c:64 * k + 16 * (c + 1)] for k in range(8)],
            axis=1)


def _tc_l2_body(a_ref, h_ref, kwr2_ref, kwo2_ref, kw2_ref, bias2_ref,
                b2t_ref, loc_ref, scale_ref, accs):
    i = pl.program_id(0)
    a_cat = jnp.concatenate([a_ref[c] for c in range(4)], axis=1)
    h_cat = jnp.concatenate([h_ref[c] for c in range(4)], axis=1)
    h3 = jnp.tanh(jnp.dot(a_cat, kwr2_ref[...], preferred_element_type=_f32)
                  + jnp.dot(h_cat, kwo2_ref[...],
                            preferred_element_type=_f32)
                  + bias2_ref[...])
    t = jnp.tanh(jnp.dot(h3, kw2_ref[...], preferred_element_type=_f32)
                 + b2t_ref[...])
    rows = (8 * lax.broadcasted_iota(jnp.int32, (BN // 8, 128), 0)
            + lax.div(lax.broadcasted_iota(jnp.int32, (BN // 8, 128), 1), 16)
            + i * BN)
    t = jnp.where(rows < N, t, 0.0)
    ps128 = jnp.sum(t, axis=0, keepdims=True)
    ps = sum(ps128[:, 16 * k:16 * (k + 1)] for k in range(8))

    @pl.when(i == 0)
    def _init():
        accs[...] = jnp.zeros_like(accs)

    accs[...] += ps

    @pl.when(i == GRID - 1)
    def _fini():
        pooled = accs[...] / _f32(N)
        loc_ref[...] = pooled[:, :8]
        sraw = pooled[:, 8:] + _f32(BIAS)
        sp = jnp.log1p(jnp.exp(sraw))
        scale_ref[...] = jnp.maximum(sp, _f32(1e-4))


_tc_prep = pl.pallas_call(
    _tc_prep_body,
    grid=(EROWS // EB,),
    in_specs=[pl.BlockSpec((2, EB, 128), lambda i: (0, i, 0))],
    out_specs=[
        pl.BlockSpec((EB, 128), lambda i: (i, 0)),
        pl.BlockSpec((EB, 128), lambda i: (i, 0)),
        pl.BlockSpec((4, EB, 128), lambda i: (0, i, 0)),
    ],
    out_shape=[
        jax.ShapeDtypeStruct((EROWS, 128), jnp.int32),
        jax.ShapeDtypeStruct((EROWS, 128), jnp.int32),
        jax.ShapeDtypeStruct((4, EROWS, 128), jnp.int32),
    ],
)

_tc_l1 = pl.pallas_call(
    _tc_l1_body,
    grid=(GRID,),
    in_specs=[
        pl.BlockSpec((2, BN // 8, 128), lambda i: (0, i, 0)),
        pl.BlockSpec((BN // 8, 128), lambda i: (i, 0)),
        pl.BlockSpec((128, 512), lambda i: (0, 0)),
        pl.BlockSpec((128, 512), lambda i: (0, 0)),
        pl.BlockSpec((1, 512), lambda i: (0, 0)),
    ],
    out_specs=pl.BlockSpec((4, BN // 8, 128), lambda i: (0, i, 0)),
    out_shape=jax.ShapeDtypeStruct((4, NP // 8, 128), _f32),
)

_tc_l2 = pl.pallas_call(
    _tc_l2_body,
    grid=(GRID,),
    in_specs=[
        pl.BlockSpec((4, BN // 8, 128), lambda i: (0, i, 0)),
        pl.BlockSpec((4, BN // 8, 128), lambda i: (0, i, 0)),
        pl.BlockSpec((512, 512), lambda i: (0, 0)),
        pl.BlockSpec((512, 512), lambda i: (0, 0)),
        pl.BlockSpec((512, 128), lambda i: (0, 0)),
        pl.BlockSpec((1, 512), lambda i: (0, 0)),
        pl.BlockSpec((1, 128), lambda i: (0, 0)),
    ],
    out_specs=[
        pl.BlockSpec((1, 8), lambda i: (0, 0)),
        pl.BlockSpec((1, 8), lambda i: (0, 0)),
    ],
    out_shape=[
        jax.ShapeDtypeStruct((1, 8), _f32),
        jax.ShapeDtypeStruct((1, 8), _f32),
    ],
    scratch_shapes=[pltpu.VMEM((1, 16), _f32)],
)


def kernel(x, W1, b1, Wr1, br1, Wo1, Wr2, br2, Wo2, W2, b2, edge_index):
    ei3 = edge_index.astype(jnp.int32).reshape(2, E // 128, 128)
    src2d, dst2d, src4 = _tc_prep(ei3)

    xp_in = jnp.pad(x, ((0, NP - N), (0, 0)))
    xpk = jnp.concatenate(
        [xp_in, jnp.ones((NP, 1), _f32), jnp.zeros((NP, 4), _f32)],
        axis=1).reshape(NP // 8, 128)
    xpad = xpk.reshape(NP, 16)

    eye8 = jnp.eye(8, dtype=_f32)
    w1pad = jnp.concatenate(
        [W1, b1[None, :], jnp.zeros((4, 64), _f32)], axis=0)
    wa = w1pad @ Wr1
    wb = w1pad @ Wo1
    kwa = jnp.einsum("kK,im->kiKm", eye8, wa).reshape(128, 512)
    kwb = jnp.einsum("kK,im->kiKm", eye8, wb).reshape(128, 512)
    bias1 = jnp.tile(br1[None, :], (1, 8))
    kwr2 = jnp.einsum("kK,cij->ckiKj", eye8,
                      Wr2.reshape(4, 16, 64)).reshape(512, 512)
    kwo2 = jnp.einsum("kK,cij->ckiKj", eye8,
                      Wo2.reshape(4, 16, 64)).reshape(512, 512)
    kw2 = jnp.einsum("kK,jm->kjKm", eye8, W2).reshape(512, 128)
    bias2 = jnp.tile(br2[None, :], (1, 8))
    b2t = jnp.tile(b2[None, :], (1, 8))

    _sc_layer1, _sc_layer2 = _get_sc_kernels()
    pflat = _sc_layer1(xpad, src2d, dst2d)
    p2 = pflat.reshape(2, NP // 8, 128)
    h2pk = _tc_l1(p2, xpk, kwa, kwb, bias1)
    aggflat = _sc_layer2(h2pk.reshape(4 * NP, 16), src4, dst2d)
    agg = aggflat.reshape(4, NP // 8, 128)
    loc2, scale2 = _tc_l2(agg, h2pk, kwr2, kwo2, kw2, bias2, b2t)
    return loc2.reshape(8), scale2.reshape(8)
